# baseline (speedup 1.0000x reference)
import functools

import jax
import jax.numpy as jnp
from jax import lax
from jax.experimental import pallas as pl
from jax.experimental.pallas import tpu as pltpu
from jax.experimental.pallas import tpu_sc as plsc

B = 8
N = 288
LATENT = 512
D = 2 * LATENT
E = 8
K = 2
TAU = 1.5
HID = 1024
R = 8
H = 1024
OUT_DIM = R * H
L = 24

T = B * N
A = K * T
TM = 128
NT = (A + TM - 1) // TM + E
MAXP = NT * TM
NTPAD = 64

NC = 2
NS = 16
NW = NC * NS
TPW = T // NW
CT = 8

_HVIEW = D // 2


def _router_body(z_ref, emb_ref, rw_ref, rb_ref, pos_ref, gat_ref, posx_ref,
                 tm_ref, aux_ref, hv_ref):
    f32 = jnp.float32
    embt = jnp.concatenate([emb_ref[...]] * B, axis=0)
    h = jnp.concatenate([z_ref[...], embt], axis=1)
    hv_ref[...] = h
    logits = (jnp.dot(h, rw_ref[...],
                      preferred_element_type=f32) + rb_ref[...]) / TAU

    neg = jnp.float32(-1e30)
    lane = lax.broadcasted_iota(jnp.int32, (E, E), 0)
    lane2 = lax.broadcasted_iota(jnp.int32, (E, E), 1)
    tril_e = (lane <= lane2).astype(f32)

    m1 = jnp.max(logits, axis=1, keepdims=True)
    eq1 = (logits == m1)
    c1 = jnp.dot(eq1.astype(f32), tril_e, preferred_element_type=f32)
    oh1 = jnp.logical_and(eq1, c1 == 1.0)
    oh1f = oh1.astype(f32)

    masked = jnp.where(oh1, neg, logits)
    m2 = jnp.max(masked, axis=1, keepdims=True)
    eq2 = (masked == m2)
    c2 = jnp.dot(eq2.astype(f32), tril_e, preferred_element_type=f32)
    oh2 = jnp.logical_and(eq2, c2 == 1.0)
    oh2f = oh2.astype(f32)

    e2 = jnp.exp(m2 - m1)
    den = 1.0 + e2
    g1 = 1.0 / den
    g2 = e2 / den

    ex = jnp.exp(logits - m1)
    probs = ex / jnp.sum(ex, axis=1, keepdims=True)

    p_es = jnp.sum(probs.reshape(B, 3, N // 3, E), axis=(0, 2)) / 768.0
    ge = oh1f * g1 + oh2f * g2
    f_es = jnp.sum(ge.reshape(B, 3, N // 3, E), axis=(0, 2)) / 768.0
    raw = jnp.float32(E) * jnp.sum(p_es * f_es, axis=1, keepdims=True)
    aux_ref[...] = jnp.sum(jnp.maximum(raw - 1.0, 0.0), axis=0, keepdims=True)

    onehot = jnp.concatenate([oh1f, oh2f], axis=0)
    cnt = jnp.sum(onehot, axis=0, keepdims=True)
    cnt_i = cnt.astype(jnp.int32)
    pc_i = ((cnt_i + (TM - 1)) // TM) * TM
    strict = (lane < lane2).astype(f32)
    po = jnp.dot(pc_i.astype(f32), strict,
                 preferred_element_type=f32)
    po_i = po.astype(jnp.int32)

    rows = lax.broadcasted_iota(jnp.int32, (TM, TM), 0)
    cols = lax.broadcasted_iota(jnp.int32, (TM, TM), 1)
    tril_m = (rows >= cols).astype(f32)
    nblk = A // TM
    blk_sum = jnp.sum(onehot.reshape(nblk, TM, E), axis=1)
    bi = lax.broadcasted_iota(jnp.int32, (nblk, nblk), 0)
    bj = lax.broadcasted_iota(jnp.int32, (nblk, nblk), 1)
    blk_pref = jnp.dot((bi > bj).astype(f32), blk_sum,
                       preferred_element_type=f32)
    pieces = []
    for i in range(nblk):
        seg = onehot[i * TM:(i + 1) * TM]
        within = jnp.dot(tril_m, seg, preferred_element_type=f32)
        pieces.append(within + blk_pref[i:i + 1])
    cum = jnp.concatenate(pieces, axis=0)

    pos_f = jnp.sum(onehot * (cum + po), axis=1, keepdims=True) - 1.0
    pos_i = pos_f.astype(jnp.int32)
    pos_ref[...] = pos_i
    gat_ref[...] = jnp.concatenate([g1, g2], axis=0)
    p0 = pos_i[0:T] * 2
    p1 = pos_i[T:A] * 2
    posx_ref[...] = jnp.concatenate([p0, p0 + 1, p1, p1 + 1], axis=0)

    mstart = lax.broadcasted_iota(jnp.int32, (1, NTPAD), 1) * TM
    te = jnp.full((1, NTPAD), -1, jnp.int32)
    for e in range(E):
        lo = lax.slice(po_i, (0, e), (1, e + 1))
        w = lax.slice(pc_i, (0, e), (1, e + 1))
        hit = jnp.logical_and(mstart >= lo, mstart < lo + w)
        te = jnp.where(hit, e, te)
    tm_ref[...] = te


def _run_router(z2, emb, rW, rb):
    return pl.pallas_call(
        _router_body,
        out_shape=(
            jax.ShapeDtypeStruct((A, 1), jnp.int32),
            jax.ShapeDtypeStruct((A, 1), jnp.float32),
            jax.ShapeDtypeStruct((2 * A, 1), jnp.int32),
            jax.ShapeDtypeStruct((1, NTPAD), jnp.int32),
            jax.ShapeDtypeStruct((1, 1), jnp.float32),
            jax.ShapeDtypeStruct((T, D), jnp.float32),
        ),
    )(z2, emb, rW, rb.reshape(1, E))


def _dispatch_body(h_hbm, pos_hbm, gat_hbm, x_hbm, rg_hbm,
                   idx_v, rows_v, gat_v, sem):
    c = lax.axis_index("c")
    s = lax.axis_index("s")
    wid = s * NC + c
    tbase = wid * TPW
    for j in range(K):
        abase = j * T + tbase
        pltpu.sync_copy(pos_hbm.at[pl.ds(abase, TPW)], idx_v)
        pltpu.sync_copy(h_hbm.at[pl.ds(tbase, TPW)], rows_v)
        pltpu.sync_copy(gat_hbm.at[pl.ds(abase, TPW)], gat_v)
        pltpu.async_copy(rows_v, x_hbm.at[idx_v], sem).wait()
        pltpu.async_copy(gat_v, rg_hbm.at[idx_v], sem).wait()


def _run_dispatch(h_view, pos1d, gat1d):
    mesh = plsc.VectorSubcoreMesh(core_axis_name="c", subcore_axis_name="s")
    fn = functools.partial(
        pl.kernel,
        out_type=(
            jax.ShapeDtypeStruct((MAXP, D), jnp.float32),
            jax.ShapeDtypeStruct((MAXP,), jnp.float32),
        ),
        mesh=mesh,
        scratch_types=[
            pltpu.VMEM((TPW,), jnp.int32),
            pltpu.VMEM((TPW, D), jnp.float32),
            pltpu.VMEM((TPW,), jnp.float32),
            pltpu.SemaphoreType.DMA,
        ],
    )(_dispatch_body)
    return fn(h_view, pos1d, gat1d)


OUTH = OUT_DIM // 2
OUTHW = OUTH // 2
OUTQ = OUT_DIM // 4


def _gemm_body(te_ref, x_ref, w1_ref, w2_ref, b1_ref, b2_ref, rg_ref, y_ref,
               a_all):
    hh = pl.program_id(0)
    m = pl.program_id(1)

    @pl.when(te_ref[m] >= 0)
    def _():
        @pl.when(hh == 0)
        def _():
            a = jnp.dot(x_ref[...], w1_ref[0],
                        preferred_element_type=jnp.float32) + b1_ref[0]
            a = a * (1.0 / (1.0 + jnp.exp(-a)))
            a_all[pl.ds(m * TM, TM), :] = a.astype(jnp.bfloat16)

        a = a_all[pl.ds(m * TM, TM), :].astype(jnp.float32)
        y = jnp.dot(a, w2_ref[0],
                    preferred_element_type=jnp.float32) + b2_ref[0]
        y_ref[...] = y * rg_ref[...]


def _run_gemm(tile_meta, x_view, W1, W2v, b1, b2v, rg_col):
    grid_spec = pltpu.PrefetchScalarGridSpec(
        num_scalar_prefetch=1,
        grid=(4, NT),
        in_specs=[
            pl.BlockSpec((TM, D), lambda h, m, te: (m, 0)),
            pl.BlockSpec((1, D, HID),
                         lambda h, m, te: (jnp.maximum(te[m], 0), 0, 0)),
            pl.BlockSpec((1, HID, OUTQ),
                         lambda h, m, te: (jnp.maximum(te[m], 0), 0, h)),
            pl.BlockSpec((1, 1, HID),
                         lambda h, m, te: (jnp.maximum(te[m], 0), 0, 0)),
            pl.BlockSpec((1, 1, OUTQ),
                         lambda h, m, te: (jnp.maximum(te[m], 0), 0, h)),
            pl.BlockSpec((TM, 1), lambda h, m, te: (m, 0)),
        ],
        out_specs=pl.BlockSpec((TM, OUTQ), lambda h, m, te: (m, h)),
        scratch_shapes=[pltpu.VMEM((MAXP, HID), jnp.bfloat16)],
    )
    return pl.pallas_call(
        _gemm_body,
        grid_spec=grid_spec,
        out_shape=jax.ShapeDtypeStruct((MAXP, OUT_DIM), jnp.float32),
    )(tile_meta, x_view, W1, W2v, b1, b2v, rg_col)


def _combine_body(yh_hbm, posx_hbm, out_hbm, idx_v, y_v, acc_v, sem):
    c = lax.axis_index("c")
    s = lax.axis_index("s")
    wid = s * NC + c
    tbase = wid * TPW
    for ci in range(TPW // CT):
        off = tbase + ci * CT
        for h in range(2):
            pltpu.sync_copy(posx_hbm.at[pl.ds(h * T + off, CT)], idx_v)
            pltpu.async_copy(yh_hbm.at[idx_v], acc_v, sem).wait()
            pltpu.sync_copy(posx_hbm.at[pl.ds((2 + h) * T + off, CT)], idx_v)
            pltpu.async_copy(yh_hbm.at[idx_v], y_v, sem).wait()

            def add_block(v):
                for r in range(CT):
                    sl = (r, pl.ds(v * 16, 16))
                    acc_v[sl] = acc_v[sl] + y_v[sl]

            lax.fori_loop(0, OUTH // 16, lambda v, _: (add_block(v), 0)[1], 0)
            pltpu.sync_copy(
                acc_v,
                out_hbm.at[pl.ds(off, CT), pl.ds(h * OUTH, OUTH)])


def _run_combine(y, posx1d):
    yh = y.reshape(2 * MAXP, OUTH)
    mesh = plsc.VectorSubcoreMesh(core_axis_name="c", subcore_axis_name="s")
    fn = functools.partial(
        pl.kernel,
        out_type=jax.ShapeDtypeStruct((T, OUT_DIM), jnp.float32),
        mesh=mesh,
        scratch_types=[
            pltpu.VMEM((CT,), jnp.int32),
            pltpu.VMEM((CT, OUTH), jnp.float32),
            pltpu.VMEM((CT, OUTH), jnp.float32),
            pltpu.SemaphoreType.DMA,
        ],
    )(_combine_body)
    return fn(yh, posx1d)


def kernel(z, emb, rW, rb, W1, b1, W2, b2):
    pos, gat, posx, tile_meta, aux11, h_view = _run_router(
        z.reshape(T, LATENT), emb, rW, rb)

    pos1d = pos.reshape(A)
    gat1d = gat.reshape(A)

    x_view, row_gate = _run_dispatch(h_view, pos1d, gat1d)

    y = _run_gemm(tile_meta.reshape(NTPAD), x_view,
                  W1, W2, b1.reshape(E, 1, HID), b2.reshape(E, 1, OUT_DIM),
                  row_gate.reshape(MAXP, 1))

    out_flat = _run_combine(y, posx.reshape(2 * A))

    out6 = out_flat.reshape(B, 3, L, 4, R, H)
    res = []
    for i in range(3):
        t = out6[:, i]
        res.append(t[:, :, 0])
        res.append(jnp.transpose(t[:, :, 1], (0, 1, 3, 2)))
        res.append(t[:, :, 2])
        res.append(jnp.transpose(t[:, :, 3], (0, 1, 3, 2)))
    return (*res, aux11.reshape(()))


# speedup vs baseline: 1.2121x; 1.2121x over previous
import functools

import jax
import jax.numpy as jnp
from jax import lax
from jax.experimental import pallas as pl
from jax.experimental.pallas import tpu as pltpu
from jax.experimental.pallas import tpu_sc as plsc

B = 8
N = 288
LATENT = 512
D = 2 * LATENT
E = 8
K = 2
TAU = 1.5
HID = 1024
R = 8
H = 1024
OUT_DIM = R * H
L = 24

T = B * N
A = K * T
TM = 128
NT = (A + TM - 1) // TM + E
MAXP = NT * TM
NTPAD = 64

NC = 2
NS = 16
NW = NC * NS
TPW = T // NW
CT = 8

_HVIEW = D // 2


def _router_body(z_ref, emb_ref, rw_ref, rb_ref, pos_ref, gat_ref, posx_ref,
                 tm_ref, aux_ref, hv_ref):
    f32 = jnp.float32
    embt = jnp.concatenate([emb_ref[...]] * B, axis=0)
    h = jnp.concatenate([z_ref[...], embt], axis=1)
    hv_ref[...] = h
    logits = (jnp.dot(h, rw_ref[...],
                      preferred_element_type=f32) + rb_ref[...]) / TAU

    neg = jnp.float32(-1e30)
    lane = lax.broadcasted_iota(jnp.int32, (E, E), 0)
    lane2 = lax.broadcasted_iota(jnp.int32, (E, E), 1)
    tril_e = (lane <= lane2).astype(f32)

    m1 = jnp.max(logits, axis=1, keepdims=True)
    eq1 = (logits == m1)
    c1 = jnp.dot(eq1.astype(f32), tril_e, preferred_element_type=f32)
    oh1 = jnp.logical_and(eq1, c1 == 1.0)
    oh1f = oh1.astype(f32)

    masked = jnp.where(oh1, neg, logits)
    m2 = jnp.max(masked, axis=1, keepdims=True)
    eq2 = (masked == m2)
    c2 = jnp.dot(eq2.astype(f32), tril_e, preferred_element_type=f32)
    oh2 = jnp.logical_and(eq2, c2 == 1.0)
    oh2f = oh2.astype(f32)

    e2 = jnp.exp(m2 - m1)
    den = 1.0 + e2
    g1 = 1.0 / den
    g2 = e2 / den

    ex = jnp.exp(logits - m1)
    probs = ex / jnp.sum(ex, axis=1, keepdims=True)

    p_es = jnp.sum(probs.reshape(B, 3, N // 3, E), axis=(0, 2)) / 768.0
    ge = oh1f * g1 + oh2f * g2
    f_es = jnp.sum(ge.reshape(B, 3, N // 3, E), axis=(0, 2)) / 768.0
    raw = jnp.float32(E) * jnp.sum(p_es * f_es, axis=1, keepdims=True)
    aux_ref[...] = jnp.sum(jnp.maximum(raw - 1.0, 0.0), axis=0, keepdims=True)

    onehot = jnp.concatenate([oh1f, oh2f], axis=0)
    cnt = jnp.sum(onehot, axis=0, keepdims=True)
    cnt_i = cnt.astype(jnp.int32)
    pc_i = ((cnt_i + (TM - 1)) // TM) * TM
    strict = (lane < lane2).astype(f32)
    po = jnp.dot(pc_i.astype(f32), strict,
                 preferred_element_type=f32)
    po_i = po.astype(jnp.int32)

    rows = lax.broadcasted_iota(jnp.int32, (TM, TM), 0)
    cols = lax.broadcasted_iota(jnp.int32, (TM, TM), 1)
    tril_m = (rows >= cols).astype(f32)
    nblk = A // TM
    blk_sum = jnp.sum(onehot.reshape(nblk, TM, E), axis=1)
    bi = lax.broadcasted_iota(jnp.int32, (nblk, nblk), 0)
    bj = lax.broadcasted_iota(jnp.int32, (nblk, nblk), 1)
    blk_pref = jnp.dot((bi > bj).astype(f32), blk_sum,
                       preferred_element_type=f32)
    pieces = []
    for i in range(nblk):
        seg = onehot[i * TM:(i + 1) * TM]
        within = jnp.dot(tril_m, seg, preferred_element_type=f32)
        pieces.append(within + blk_pref[i:i + 1])
    cum = jnp.concatenate(pieces, axis=0)

    pos_f = jnp.sum(onehot * (cum + po), axis=1, keepdims=True) - 1.0
    pos_i = pos_f.astype(jnp.int32)
    pos_ref[...] = pos_i
    gat_ref[...] = jnp.concatenate([g1, g2], axis=0)
    p0 = pos_i[0:T] * 2
    p1 = pos_i[T:A] * 2
    posx_ref[...] = jnp.concatenate([p0, p0 + 1, p1, p1 + 1], axis=0)

    mstart = lax.broadcasted_iota(jnp.int32, (1, NTPAD), 1) * TM
    te = jnp.full((1, NTPAD), -1, jnp.int32)
    for e in range(E):
        lo = lax.slice(po_i, (0, e), (1, e + 1))
        w = lax.slice(pc_i, (0, e), (1, e + 1))
        hit = jnp.logical_and(mstart >= lo, mstart < lo + w)
        te = jnp.where(hit, e, te)
    tm_ref[...] = te


def _run_router(z2, emb, rW, rb):
    return pl.pallas_call(
        _router_body,
        out_shape=(
            jax.ShapeDtypeStruct((A, 1), jnp.int32),
            jax.ShapeDtypeStruct((A, 1), jnp.float32),
            jax.ShapeDtypeStruct((2 * A, 1), jnp.int32),
            jax.ShapeDtypeStruct((1, NTPAD), jnp.int32),
            jax.ShapeDtypeStruct((1, 1), jnp.float32),
            jax.ShapeDtypeStruct((T, D), jnp.float32),
        ),
    )(z2, emb, rW, rb.reshape(1, E))


def _dispatch_body(h_hbm, pos_hbm, gat_hbm, x_hbm, rg_hbm,
                   idx_v, rows_v, gat_v, sem):
    c = lax.axis_index("c")
    s = lax.axis_index("s")
    wid = s * NC + c
    tbase = wid * TPW
    for j in range(K):
        abase = j * T + tbase
        pltpu.sync_copy(pos_hbm.at[pl.ds(abase, TPW)], idx_v)
        pltpu.sync_copy(h_hbm.at[pl.ds(tbase, TPW)], rows_v)
        pltpu.sync_copy(gat_hbm.at[pl.ds(abase, TPW)], gat_v)
        pltpu.async_copy(rows_v, x_hbm.at[idx_v], sem).wait()
        pltpu.async_copy(gat_v, rg_hbm.at[idx_v], sem).wait()


def _run_dispatch(h_view, pos1d, gat1d):
    mesh = plsc.VectorSubcoreMesh(core_axis_name="c", subcore_axis_name="s")
    fn = functools.partial(
        pl.kernel,
        out_type=(
            jax.ShapeDtypeStruct((MAXP, D), jnp.float32),
            jax.ShapeDtypeStruct((MAXP,), jnp.float32),
        ),
        mesh=mesh,
        scratch_types=[
            pltpu.VMEM((TPW,), jnp.int32),
            pltpu.VMEM((TPW, D), jnp.float32),
            pltpu.VMEM((TPW,), jnp.float32),
            pltpu.SemaphoreType.DMA,
        ],
    )(_dispatch_body)
    return fn(h_view, pos1d, gat1d)


OUTH = OUT_DIM // 2
OUTHW = OUTH // 2
OUTQ = OUT_DIM // 4


def _gemm_body(te_ref, x_ref, w1_ref, w2_ref, b1_ref, b2_ref, rg_ref, y_ref,
               a_all):
    hh = pl.program_id(0)
    m = pl.program_id(1)

    @pl.when(te_ref[m] >= 0)
    def _():
        @pl.when(hh == 0)
        def _():
            a = jnp.dot(x_ref[...], w1_ref[0],
                        preferred_element_type=jnp.float32) + b1_ref[0]
            a = a * (1.0 / (1.0 + jnp.exp(-a)))
            a_all[pl.ds(m * TM, TM), :] = a.astype(jnp.bfloat16)

        a = a_all[pl.ds(m * TM, TM), :].astype(jnp.float32)
        y = jnp.dot(a, w2_ref[0],
                    preferred_element_type=jnp.float32) + b2_ref[0]
        y_ref[...] = y * rg_ref[...]


def _run_gemm(tile_meta, x_view, W1, W2v, b1, b2v, rg_col):
    grid_spec = pltpu.PrefetchScalarGridSpec(
        num_scalar_prefetch=1,
        grid=(2, NT),
        in_specs=[
            pl.BlockSpec((TM, D), lambda h, m, te: (m, 0)),
            pl.BlockSpec((1, D, HID),
                         lambda h, m, te: (jnp.maximum(te[m], 0), 0, 0)),
            pl.BlockSpec((1, HID, OUTH),
                         lambda h, m, te: (jnp.maximum(te[m], 0), 0, h)),
            pl.BlockSpec((1, 1, HID),
                         lambda h, m, te: (jnp.maximum(te[m], 0), 0, 0)),
            pl.BlockSpec((1, 1, OUTH),
                         lambda h, m, te: (jnp.maximum(te[m], 0), 0, h)),
            pl.BlockSpec((TM, 1), lambda h, m, te: (m, 0)),
        ],
        out_specs=pl.BlockSpec((TM, OUTH), lambda h, m, te: (m, h)),
        scratch_shapes=[pltpu.VMEM((MAXP, HID), jnp.bfloat16)],
    )
    return pl.pallas_call(
        _gemm_body,
        grid_spec=grid_spec,
        out_shape=jax.ShapeDtypeStruct((MAXP, OUT_DIM), jnp.float32),
    )(tile_meta, x_view, W1, W2v, b1, b2v, rg_col)


CQ = 4


def _combine_body(y_hbm, pos_hbm, pp_hbm, out_hbm,
                  idx0_v, idx1_v, acc_v, y_v, sem, sem2):
    c = lax.axis_index("c")
    s = lax.axis_index("s")
    wid = s * NC + c
    tbase = wid * TPW
    for ci in range(TPW // CT):
        off = tbase + ci * CT
        g0 = off // CQ
        pltpu.sync_copy(pos_hbm.at[pl.ds(off, CT)], idx0_v)
        d0 = pltpu.async_copy(y_hbm.at[idx0_v], acc_v, sem)
        pltpu.sync_copy(pp_hbm.at[pl.ds(8 * (T // CQ + g0), 8)], idx1_v)
        d0.wait()
        e0 = pltpu.async_copy(y_hbm.at[idx1_v.at[pl.ds(0, CQ)]], y_v, sem2)
        e0.wait()

        def add_lo(v):
            for r in range(CQ):
                sl = (r, pl.ds(v * 16, 16))
                acc_v[sl] = acc_v[sl] + y_v[sl]

        lax.fori_loop(0, OUT_DIM // 16, lambda v, _: (add_lo(v), 0)[1], 0)
        pltpu.sync_copy(pp_hbm.at[pl.ds(8 * (T // CQ + g0 + 1), 8)], idx1_v)
        e1 = pltpu.async_copy(y_hbm.at[idx1_v.at[pl.ds(0, CQ)]], y_v, sem2)
        e1.wait()

        def add_hi(v):
            for r in range(CQ):
                sl = (CQ + r, pl.ds(v * 16, 16))
                slr = (r, pl.ds(v * 16, 16))
                acc_v[sl] = acc_v[sl] + y_v[slr]

        lax.fori_loop(0, OUT_DIM // 16, lambda v, _: (add_hi(v), 0)[1], 0)
        pltpu.sync_copy(acc_v, out_hbm.at[pl.ds(off, CT)])


def _run_combine(y, pos1d):
    pos_pad = jnp.pad(pos1d.reshape(A // CQ, CQ),
                      ((0, 0), (0, 8 - CQ))).reshape(2 * A)
    mesh = plsc.VectorSubcoreMesh(core_axis_name="c", subcore_axis_name="s")
    fn = functools.partial(
        pl.kernel,
        out_type=jax.ShapeDtypeStruct((T, OUT_DIM), jnp.float32),
        mesh=mesh,
        scratch_types=[
            pltpu.VMEM((CT,), jnp.int32),
            pltpu.VMEM((8,), jnp.int32),
            pltpu.VMEM((CT, OUT_DIM), jnp.float32),
            pltpu.VMEM((CQ, OUT_DIM), jnp.float32),
            pltpu.SemaphoreType.DMA,
            pltpu.SemaphoreType.DMA,
        ],
    )(_combine_body)
    return fn(y, pos1d, pos_pad)


def kernel(z, emb, rW, rb, W1, b1, W2, b2):
    pos, gat, posx, tile_meta, aux11, h_view = _run_router(
        z.reshape(T, LATENT), emb, rW, rb)

    pos1d = pos.reshape(A)
    gat1d = gat.reshape(A)

    x_view, row_gate = _run_dispatch(h_view, pos1d, gat1d)

    y = _run_gemm(tile_meta.reshape(NTPAD), x_view,
                  W1, W2, b1.reshape(E, 1, HID), b2.reshape(E, 1, OUT_DIM),
                  row_gate.reshape(MAXP, 1))

    out_flat = _run_combine(y, pos1d)

    out6 = out_flat.reshape(B, 3, L, 4, R, H)
    res = []
    for i in range(3):
        t = out6[:, i]
        res.append(t[:, :, 0])
        res.append(jnp.transpose(t[:, :, 1], (0, 1, 3, 2)))
        res.append(t[:, :, 2])
        res.append(jnp.transpose(t[:, :, 3], (0, 1, 3, 2)))
    return (*res, aux11.reshape(()))
